# baseline (speedup 1.0000x reference)
import jax
import jax.numpy as jnp
from jax import lax
from jax.experimental import pallas as pl
from jax.experimental.pallas import tpu as pltpu
from jax.experimental.pallas import tpu_sc as plsc

N = 10000
E = 160000
G = 256
HID = 128

NC = 2
NS = 16
NW = NC * NS

NPAD = 10240
RPT = NPAD // NS
EPAD = 163840
C = 128
EPT = EPAD // NW
NCHUNK = EPT // C

BR = 2048
NBLK = NPAD // BR

_MESH = plsc.VectorSubcoreMesh(
    core_axis_name="c", subcore_axis_name="s", num_cores=NC, num_subcores=NS)
_SC_PARAMS = pltpu.CompilerParams(use_tc_tiling_on_sc=False)


NBUF = 4

FAST = 0
NCKF = 76
NCKS = 4
NCKMAX = NCKF


def _make_deg_kernel():
    def body(dst2, w2, zf, out, dstv, wv, obuf, accw, accc, *sems):
        semw = sems[0:NBUF]
        semc = sems[NBUF:2 * NBUF]
        c = lax.axis_index("c")
        s = lax.axis_index("s")
        wid = c * NS + s
        pltpu.sync_copy(dst2.at[pl.ds(wid * NCHUNK, NCHUNK)], dstv)
        pltpu.sync_copy(w2.at[pl.ds(wid * NCHUNK, NCHUNK)], wv)
        pltpu.sync_copy(zf, accw.at[pl.ds(s * RPT, RPT)])
        pltpu.sync_copy(zf, accc.at[pl.ds(s * RPT, RPT)])
        for g in range(C // 16):
            obuf[pl.ds(g * 16, 16)] = jnp.ones((16,), jnp.float32)
        plsc.subcore_barrier()

        def outer(i2, carry):
            for b in range(NBUF):
                i = i2 * NBUF + b

                @pl.when(i >= NBUF)
                def _():
                    pltpu.make_async_copy(wv.at[i - NBUF], accw.at[dstv.at[i - NBUF]], semw[b]).wait()
                    pltpu.make_async_copy(obuf, accc.at[dstv.at[i - NBUF]], semc[b]).wait()

                pltpu.async_copy(wv.at[i], accw.at[dstv.at[i]], semw[b], add=True)
                pltpu.async_copy(obuf, accc.at[dstv.at[i]], semc[b], add=True)
            return carry

        lax.fori_loop(0, NCHUNK // NBUF, outer, 0)
        for j in range(NCHUNK - NBUF, NCHUNK):
            b = j % NBUF
            pltpu.make_async_copy(wv.at[j], accw.at[dstv.at[j]], semw[b]).wait()
            pltpu.make_async_copy(obuf, accc.at[dstv.at[j]], semc[b]).wait()
        plsc.subcore_barrier()
        pltpu.sync_copy(accw.at[pl.ds(s * RPT, RPT)], out.at[pl.ds(c * 2 * NPAD + s * RPT, RPT)])
        pltpu.sync_copy(accc.at[pl.ds(s * RPT, RPT)], out.at[pl.ds(c * 2 * NPAD + NPAD + s * RPT, RPT)])

    return pl.kernel(
        body,
        out_type=jax.ShapeDtypeStruct((NC * 2 * NPAD,), jnp.float32),
        mesh=_MESH,
        scratch_types=[
            pltpu.VMEM((NCHUNK, C), jnp.int32),
            pltpu.VMEM((NCHUNK, C), jnp.float32),
            pltpu.VMEM((C,), jnp.float32),
            pltpu.VMEM_SHARED((NPAD,), jnp.float32),
            pltpu.VMEM_SHARED((NPAD,), jnp.float32),
        ] + [pltpu.SemaphoreType.DMA] * (2 * NBUF),
        compiler_params=_SC_PARAMS,
    )


def _make_edge128_kernel():
    F = HID
    NB = 2

    def body(src2, dst2, wrep, g, out, dstv, *scr):
        srcb = scr[0:NB]
        wrepv = scr[NB:2 * NB]
        rows = scr[2 * NB:3 * NB]
        acc = scr[3 * NB]
        semi = scr[3 * NB + 1:3 * NB + 1 + NB]
        semg = scr[3 * NB + 1 + NB:3 * NB + 1 + 2 * NB]
        semw = scr[3 * NB + 1 + 2 * NB:3 * NB + 1 + 3 * NB]
        sems = scr[3 * NB + 1 + 3 * NB:3 * NB + 1 + 4 * NB]
        c = lax.axis_index("c")
        s = lax.axis_index("s")
        fast = c == FAST
        nck = jnp.where(fast, NCKF, NCKS)
        cbase = jnp.where(fast, s * NCKF, NS * NCKF + s * NCKS)
        pltpu.sync_copy(dst2.at[pl.ds(cbase, NCKS)], dstv.at[pl.ds(0, NCKS)])
        pltpu.sync_copy(dst2.at[pl.ds(jnp.where(fast, cbase + NCKS, 0), NCKF - NCKS)],
                        dstv.at[pl.ds(NCKS, NCKF - NCKS)])
        wbase = cbase * 16

        def fire_idx(i, b):
            pltpu.async_copy(src2.at[cbase + i], srcb[b], semi[b])

        def wait_idx(i, b):
            pltpu.make_async_copy(src2.at[cbase + i], srcb[b], semi[b]).wait()

        def fire_gather(i, b):
            pltpu.async_copy(g.at[srcb[b]], rows[b], semg[b])
            pltpu.async_copy(wrep.at[pl.ds(wbase + i * 16, 16)], wrepv[b], semw[b])

        def wait_gather(i, b):
            pltpu.make_async_copy(g.at[srcb[b]], rows[b], semg[b]).wait()
            pltpu.make_async_copy(wrep.at[pl.ds(wbase + i * 16, 16)], wrepv[b], semw[b]).wait()

        def fire_scatter(i, b):
            pltpu.async_copy(rows[b], acc.at[dstv.at[i]], sems[b], add=True)

        def wait_scatter(i, b):
            pltpu.make_async_copy(rows[b], acc.at[dstv.at[i]], sems[b]).wait()

        def zbody(r, zcarry):
            for j in range(F // 16):
                rows[1][r, pl.ds(j * 16, 16)] = jnp.zeros((16,), jnp.float32)
            return zcarry
        lax.fori_loop(0, C, zbody, 0)
        for r5 in range(RPT // C):
            pltpu.sync_copy(rows[1], acc.at[pl.ds(s * RPT + r5 * C, C)])

        fire_idx(0, 0)
        fire_idx(1, 1)
        wait_idx(0, 0)
        fire_gather(0, 0)
        plsc.subcore_barrier()
        fire_scatter(0, 1)

        def outer(i2, carry):
            for b in range(NB):
                i = i2 * NB + b
                bn = (b + 1) % NB
                wait_gather(i, b)
                fire_idx(lax.rem(i + 2, nck), b)
                wait_scatter(lax.rem(i + nck - 1, nck), bn)
                wait_idx(lax.rem(i + 1, nck), bn)
                fire_gather(lax.rem(i + 1, nck), bn)

                def ebody(e, ecarry):
                    wb = wrepv[b][e >> 3, pl.ds((e & 7) * 16, 16)]
                    for j in range(F // 16):
                        rows[b][e, pl.ds(j * 16, 16)] = rows[b][e, pl.ds(j * 16, 16)] * wb
                    return ecarry

                lax.fori_loop(0, C, ebody, 0)
                fire_scatter(i, b)
            return carry

        lax.fori_loop(0, jnp.where(fast, NCKF // NB, NCKS // NB), outer, 0)
        wait_scatter(nck - 1, 1)
        wait_gather(0, 0)
        wait_idx(1, 1)
        plsc.subcore_barrier()
        pltpu.sync_copy(acc.at[pl.ds(s * RPT, RPT)], out.at[c, pl.ds(s * RPT, RPT)])

    return pl.kernel(
        body,
        out_type=jax.ShapeDtypeStruct((NC, NPAD, F), jnp.float32),
        mesh=_MESH,
        scratch_types=[
            pltpu.VMEM((NCKMAX, C), jnp.int32),
        ] + [pltpu.VMEM((C,), jnp.int32)] * NB
          + [pltpu.VMEM((16, 128), jnp.float32)] * NB
          + [pltpu.VMEM((C, F), jnp.float32)] * NB
          + [pltpu.VMEM_SHARED((NPAD, F), jnp.float32)]
          + [pltpu.SemaphoreType.DMA] * (4 * NB),
        compiler_params=_SC_PARAMS,
    )


def _make_edge1_kernel():
    def body(src2, dst2, g, zf, out, srcv, dstv, *scr):
        rows = scr[0:NBUF]
        acc = scr[NBUF]
        semg = scr[NBUF + 1:NBUF + 1 + NBUF]
        sems = scr[NBUF + 1 + NBUF:NBUF + 1 + 2 * NBUF]
        c = lax.axis_index("c")
        s = lax.axis_index("s")
        wid = c * NS + s
        pltpu.sync_copy(src2.at[pl.ds(wid * NCHUNK, NCHUNK)], srcv)
        pltpu.sync_copy(dst2.at[pl.ds(wid * NCHUNK, NCHUNK)], dstv)
        pltpu.sync_copy(zf, acc.at[pl.ds(s * RPT, RPT)])

        pltpu.async_copy(g.at[srcv.at[0]], rows[0], semg[0])
        pltpu.async_copy(g.at[srcv.at[1]], rows[1], semg[1])
        plsc.subcore_barrier()

        def outer(i2, carry):
            for b in range(NBUF):
                i = i2 * NBUF + b
                bn = (b + 2) % NBUF

                @pl.when(i + 2 < NCHUNK)
                def _():
                    @pl.when(i >= 2)
                    def _():
                        pltpu.make_async_copy(rows[bn], acc.at[dstv.at[i - 2]], sems[bn]).wait()
                    pltpu.async_copy(g.at[srcv.at[i + 2]], rows[bn], semg[bn])

                pltpu.make_async_copy(g.at[srcv.at[i]], rows[b], semg[b]).wait()
                pltpu.async_copy(rows[b], acc.at[dstv.at[i]], sems[b], add=True)
            return carry

        lax.fori_loop(0, NCHUNK // NBUF, outer, 0)
        for j in range(NCHUNK - NBUF, NCHUNK):
            b = j % NBUF
            pltpu.make_async_copy(rows[b], acc.at[dstv.at[j]], sems[b]).wait()
        plsc.subcore_barrier()
        pltpu.sync_copy(acc.at[pl.ds(s * RPT, RPT)], out.at[pl.ds(c * NPAD + s * RPT, RPT)])

    return pl.kernel(
        body,
        out_type=jax.ShapeDtypeStruct((NC * NPAD,), jnp.float32),
        mesh=_MESH,
        scratch_types=[
            pltpu.VMEM((NCHUNK, C), jnp.int32),
            pltpu.VMEM((NCHUNK, C), jnp.int32),
        ] + [pltpu.VMEM((C,), jnp.float32)] * NBUF
          + [pltpu.VMEM_SHARED((NPAD,), jnp.float32)]
          + [pltpu.SemaphoreType.DMA] * (2 * NBUF),
        compiler_params=_SC_PARAMS,
    )


_deg_kernel = _make_deg_kernel()
_edge128 = _make_edge128_kernel()
_edge1 = _make_edge1_kernel()


def _tc_prep(xp, obsp, Wx, We, be, W0p, dw0, dw1, dc0, dc1):
    def body(x_ref, o_ref, wx_ref, we_ref, be_ref, w0_ref,
             dw0_ref, dw1_ref, dc0_ref, dc1_ref,
             u0_ref, g0_ref, dw_ref, d1_ref):
        h0 = (jnp.dot(x_ref[...], wx_ref[...], preferred_element_type=jnp.float32)
              + jnp.dot(o_ref[...], we_ref[...], preferred_element_type=jnp.float32)
              + be_ref[...])
        degw = dw0_ref[...] + dw1_ref[...] + 1.0
        deg1 = dc0_ref[...] + dc1_ref[...] + 1.0
        dw = lax.rsqrt(degw)
        d1 = lax.rsqrt(deg1)
        u0 = jnp.dot(h0, w0_ref[...], preferred_element_type=jnp.float32)
        u0_ref[...] = u0
        g0_ref[...] = u0 * dw
        dw_ref[...] = dw
        d1_ref[...] = d1

    return pl.pallas_call(
        body,
        out_shape=[
            jax.ShapeDtypeStruct((NPAD, HID), jnp.float32),
            jax.ShapeDtypeStruct((NPAD, HID), jnp.float32),
            jax.ShapeDtypeStruct((NPAD, 1), jnp.float32),
            jax.ShapeDtypeStruct((NPAD, 1), jnp.float32),
        ],
    )(xp, obsp, Wx, We, be, W0p, dw0, dw1, dc0, dc1)


def _tc_mid():
    def body(acc_ref, u_ref, dv_ref, b_ref, a_ref, w_ref, un_ref, gn_ref):
        acc = acc_ref[...]
        dv = dv_ref[...]
        t = dv * (acc[0] + acc[1]) + dv * dv * u_ref[...] + b_ref[...]
        a = a_ref[0, 0]
        hn = jnp.where(t >= 0, t, a * t)
        un = jnp.dot(hn, w_ref[...], preferred_element_type=jnp.float32)
        un_ref[...] = un
        gn_ref[...] = un * dv

    return pl.pallas_call(
        body,
        grid=(NBLK,),
        in_specs=[
            pl.BlockSpec((2, BR, HID), lambda i: (0, i, 0)),
            pl.BlockSpec((BR, HID), lambda i: (i, 0)),
            pl.BlockSpec((BR, 1), lambda i: (i, 0)),
            pl.BlockSpec((1, HID), lambda i: (0, 0)),
            pl.BlockSpec((1, 1), lambda i: (0, 0)),
            pl.BlockSpec((HID, HID), lambda i: (0, 0)),
        ],
        out_specs=[
            pl.BlockSpec((BR, HID), lambda i: (i, 0)),
            pl.BlockSpec((BR, HID), lambda i: (i, 0)),
        ],
        out_shape=[
            jax.ShapeDtypeStruct((NPAD, HID), jnp.float32),
            jax.ShapeDtypeStruct((NPAD, HID), jnp.float32),
        ],
    )


_tc_mid_call = _tc_mid()


def _tc_l3(acc, u3, dinvw, b3, a, W4, dinv1):
    def body(acc_ref, u_ref, dv_ref, b_ref, a_ref, w4_ref, d1_ref, u4_ref, g4_ref):
        accv = acc_ref[...]
        dv = dv_ref[...]
        t = dv * (accv[0] + accv[1]) + dv * dv * u_ref[...] + b_ref[...]
        av = a_ref[0, 0]
        h4 = jnp.where(t >= 0, t, av * t)
        u4 = jnp.dot(h4, w4_ref[...], preferred_element_type=jnp.float32)
        u4_ref[...] = u4
        g4_ref[...] = u4 * d1_ref[...]

    return pl.pallas_call(
        body,
        grid=(NBLK,),
        in_specs=[
            pl.BlockSpec((2, BR, HID), lambda i: (0, i, 0)),
            pl.BlockSpec((BR, HID), lambda i: (i, 0)),
            pl.BlockSpec((BR, 1), lambda i: (i, 0)),
            pl.BlockSpec((1, HID), lambda i: (0, 0)),
            pl.BlockSpec((1, 1), lambda i: (0, 0)),
            pl.BlockSpec((HID, 1), lambda i: (0, 0)),
            pl.BlockSpec((BR, 1), lambda i: (i, 0)),
        ],
        out_specs=[
            pl.BlockSpec((BR, 1), lambda i: (i, 0)),
            pl.BlockSpec((BR, 1), lambda i: (i, 0)),
        ],
        out_shape=[
            jax.ShapeDtypeStruct((NPAD, 1), jnp.float32),
            jax.ShapeDtypeStruct((NPAD, 1), jnp.float32),
        ],
    )(acc, u3, dinvw, b3, a, W4, dinv1)


def _tc_final(acc4, u4, dinv1, b4, ids):
    def body(acc_ref, u_ref, d1_ref, b_ref, ids_ref, out_ref, sums_ref):
        i = pl.program_id(0)

        @pl.when(i == 0)
        def _():
            sums_ref[...] = jnp.zeros((2, G), jnp.float32)

        accv = acc_ref[...]
        d1 = d1_ref[...]
        z = d1 * (accv[0] + accv[1]) + d1 * d1 * u_ref[...] + b_ref[0, 0]
        iota = lax.broadcasted_iota(jnp.int32, (1, G), 1)
        m = (ids_ref[...] == iota).astype(jnp.float32)
        sums_ref[0:1, :] += jnp.sum(m * z, axis=0, keepdims=True)
        sums_ref[1:2, :] += jnp.sum(m, axis=0, keepdims=True)

        @pl.when(i == NBLK - 1)
        def _():
            out_ref[...] = sums_ref[0:1, :] / jnp.maximum(sums_ref[1:2, :], 1.0)

    return pl.pallas_call(
        body,
        grid=(NBLK,),
        in_specs=[
            pl.BlockSpec((2, BR, 1), lambda i: (0, i, 0)),
            pl.BlockSpec((BR, 1), lambda i: (i, 0)),
            pl.BlockSpec((BR, 1), lambda i: (i, 0)),
            pl.BlockSpec((1, 1), lambda i: (0, 0)),
            pl.BlockSpec((BR, 1), lambda i: (i, 0)),
        ],
        out_specs=pl.BlockSpec((1, G), lambda i: (0, 0)),
        out_shape=jax.ShapeDtypeStruct((1, G), jnp.float32),
        scratch_shapes=[pltpu.VMEM((2, G), jnp.float32)],
    )(acc4, u4, dinv1, b4, ids)


def kernel(x, observation, edge_index, edge_weight, batch_ids,
           W_emb, b_emb, W0, b0, W1, b1, W2, b2, W3, b3, W4, b4, prelu_a):
    f32 = jnp.float32
    src = edge_index[0]
    dst = edge_index[1]
    npad_e = EPAD - E

    srcp = jnp.concatenate([src, jnp.full((npad_e,), N, jnp.int32)]).reshape(EPAD // C, C)
    dstp = jnp.concatenate([dst, jnp.full((npad_e,), N, jnp.int32)]).reshape(EPAD // C, C)
    wp = jnp.concatenate([edge_weight, jnp.zeros((npad_e,), f32)])
    w2 = wp.reshape(EPAD // C, C)
    wrep = jnp.broadcast_to(wp[:, None], (EPAD, 16)).reshape(EPAD * 16 // C, C)

    xp = jnp.pad(x, ((0, NPAD - N), (0, 0)))
    obsp = jnp.pad(observation, ((0, NPAD - N), (0, 0)))
    ids = jnp.pad(batch_ids, (0, NPAD - N), constant_values=G).reshape(NPAD, 1)

    Wx = jnp.eye(4, 16, dtype=f32)
    We = jnp.zeros((6, 16), f32).at[:, 4:10].set(W_emb)
    be = jnp.zeros((1, 16), f32).at[0, 4:10].set(b_emb)
    W0p = jnp.zeros((16, HID), f32).at[0:10, :].set(W0)

    zf1 = jnp.zeros((RPT,), f32)

    a = prelu_a.reshape(1, 1)

    degf = _deg_kernel(dstp, w2, zf1)
    dw0 = degf[0 * NPAD:1 * NPAD].reshape(NPAD, 1)
    dc0 = degf[1 * NPAD:2 * NPAD].reshape(NPAD, 1)
    dw1 = degf[2 * NPAD:3 * NPAD].reshape(NPAD, 1)
    dc1 = degf[3 * NPAD:4 * NPAD].reshape(NPAD, 1)
    u0, g0, dinvw, dinv1 = _tc_prep(xp, obsp, Wx, We, be, W0p, dw0, dw1, dc0, dc1)

    acc0 = _edge128(srcp, dstp, wrep, g0)
    u1, g1 = _tc_mid_call(acc0, u0, dinvw, b0.reshape(1, HID), a, W1)

    acc1 = _edge128(srcp, dstp, wrep, g1)
    u2, g2 = _tc_mid_call(acc1, u1, dinvw, b1.reshape(1, HID), a, W2)

    acc2 = _edge128(srcp, dstp, wrep, g2)
    u3, g3 = _tc_mid_call(acc2, u2, dinvw, b2.reshape(1, HID), a, W3)

    acc3 = _edge128(srcp, dstp, wrep, g3)
    u4, g4 = _tc_l3(acc3, u3, dinvw, b3.reshape(1, HID), a, W4, dinv1)

    acc4 = _edge1(srcp, dstp, g4.reshape(NPAD), zf1).reshape(NC, NPAD, 1)
    pooled = _tc_final(acc4, u4, dinv1, b4.reshape(1, 1), ids)
    return pooled.reshape(-1)


# speedup vs baseline: 1.0649x; 1.0649x over previous
import jax
import jax.numpy as jnp
from jax import lax
from jax.experimental import pallas as pl
from jax.experimental.pallas import tpu as pltpu
from jax.experimental.pallas import tpu_sc as plsc

N = 10000
E = 160000
G = 256
HID = 128

NC = 2
NS = 16
NW = NC * NS

NPAD = 10240
RPT = NPAD // NS
EPAD = 163840
C = 128
EPT = EPAD // NW
NCHUNK = EPT // C

BR = 2048
NBLK = NPAD // BR

_MESH = plsc.VectorSubcoreMesh(
    core_axis_name="c", subcore_axis_name="s", num_cores=NC, num_subcores=NS)
_SC_PARAMS = pltpu.CompilerParams(use_tc_tiling_on_sc=False)


NBUF = 4

FAST = 0
NCKF = 72
NCKS = 8
NCKMAX = NCKF


def _make_deg_kernel():
    def body(dst2, w2, zf, out, dstv, wv, obuf, accw, accc, *sems):
        semw = sems[0:NBUF]
        semc = sems[NBUF:2 * NBUF]
        c = lax.axis_index("c")
        s = lax.axis_index("s")
        wid = c * NS + s
        pltpu.sync_copy(dst2.at[pl.ds(wid * NCHUNK, NCHUNK)], dstv)
        pltpu.sync_copy(w2.at[pl.ds(wid * NCHUNK, NCHUNK)], wv)
        pltpu.sync_copy(zf, accw.at[pl.ds(s * RPT, RPT)])
        pltpu.sync_copy(zf, accc.at[pl.ds(s * RPT, RPT)])
        for g in range(C // 16):
            obuf[pl.ds(g * 16, 16)] = jnp.ones((16,), jnp.float32)
        plsc.subcore_barrier()

        def outer(i2, carry):
            for b in range(NBUF):
                i = i2 * NBUF + b

                @pl.when(i >= NBUF)
                def _():
                    pltpu.make_async_copy(wv.at[i - NBUF], accw.at[dstv.at[i - NBUF]], semw[b]).wait()
                    pltpu.make_async_copy(obuf, accc.at[dstv.at[i - NBUF]], semc[b]).wait()

                pltpu.async_copy(wv.at[i], accw.at[dstv.at[i]], semw[b], add=True)
                pltpu.async_copy(obuf, accc.at[dstv.at[i]], semc[b], add=True)
            return carry

        lax.fori_loop(0, NCHUNK // NBUF, outer, 0)
        for j in range(NCHUNK - NBUF, NCHUNK):
            b = j % NBUF
            pltpu.make_async_copy(wv.at[j], accw.at[dstv.at[j]], semw[b]).wait()
            pltpu.make_async_copy(obuf, accc.at[dstv.at[j]], semc[b]).wait()
        plsc.subcore_barrier()
        pltpu.sync_copy(accw.at[pl.ds(s * RPT, RPT)], out.at[pl.ds(c * 2 * NPAD + s * RPT, RPT)])
        pltpu.sync_copy(accc.at[pl.ds(s * RPT, RPT)], out.at[pl.ds(c * 2 * NPAD + NPAD + s * RPT, RPT)])

    return pl.kernel(
        body,
        out_type=jax.ShapeDtypeStruct((NC * 2 * NPAD,), jnp.float32),
        mesh=_MESH,
        scratch_types=[
            pltpu.VMEM((NCHUNK, C), jnp.int32),
            pltpu.VMEM((NCHUNK, C), jnp.float32),
            pltpu.VMEM((C,), jnp.float32),
            pltpu.VMEM_SHARED((NPAD,), jnp.float32),
            pltpu.VMEM_SHARED((NPAD,), jnp.float32),
        ] + [pltpu.SemaphoreType.DMA] * (2 * NBUF),
        compiler_params=_SC_PARAMS,
    )


def _make_edge128_kernel():
    F = HID
    NB = 2

    def body(src2, dst2, wrep, g, out, dstv, *scr):
        srcb = scr[0:NB]
        wrepv = scr[NB:2 * NB]
        rows = scr[2 * NB:3 * NB]
        acc = scr[3 * NB]
        semi = scr[3 * NB + 1:3 * NB + 1 + NB]
        semg = scr[3 * NB + 1 + NB:3 * NB + 1 + 2 * NB]
        semw = scr[3 * NB + 1 + 2 * NB:3 * NB + 1 + 3 * NB]
        sems = scr[3 * NB + 1 + 3 * NB:3 * NB + 1 + 4 * NB]
        c = lax.axis_index("c")
        s = lax.axis_index("s")
        fast = c == FAST
        nck = jnp.where(fast, NCKF, NCKS)
        cbase = jnp.where(fast, s * NCKF, NS * NCKF + s * NCKS)
        pltpu.sync_copy(dst2.at[pl.ds(cbase, NCKS)], dstv.at[pl.ds(0, NCKS)])
        pltpu.sync_copy(dst2.at[pl.ds(jnp.where(fast, cbase + NCKS, 0), NCKF - NCKS)],
                        dstv.at[pl.ds(NCKS, NCKF - NCKS)])
        wbase = cbase * 16

        def fire_idx(i, b):
            pltpu.async_copy(src2.at[cbase + i], srcb[b], semi[b])

        def wait_idx(i, b):
            pltpu.make_async_copy(src2.at[cbase + i], srcb[b], semi[b]).wait()

        def fire_gather(i, b):
            pltpu.async_copy(g.at[srcb[b]], rows[b], semg[b])
            pltpu.async_copy(wrep.at[pl.ds(wbase + i * 16, 16)], wrepv[b], semw[b])

        def wait_gather(i, b):
            pltpu.make_async_copy(g.at[srcb[b]], rows[b], semg[b]).wait()
            pltpu.make_async_copy(wrep.at[pl.ds(wbase + i * 16, 16)], wrepv[b], semw[b]).wait()

        def fire_scatter(i, b):
            pltpu.async_copy(rows[b], acc.at[dstv.at[i]], sems[b], add=True)

        def wait_scatter(i, b):
            pltpu.make_async_copy(rows[b], acc.at[dstv.at[i]], sems[b]).wait()

        def zbody(r, zcarry):
            for j in range(F // 16):
                rows[1][r, pl.ds(j * 16, 16)] = jnp.zeros((16,), jnp.float32)
            return zcarry
        lax.fori_loop(0, C, zbody, 0)
        for r5 in range(RPT // C):
            pltpu.sync_copy(rows[1], acc.at[pl.ds(s * RPT + r5 * C, C)])

        fire_idx(0, 0)
        fire_idx(1, 1)
        wait_idx(0, 0)
        fire_gather(0, 0)
        plsc.subcore_barrier()
        fire_scatter(0, 1)

        def outer(i2, carry):
            for b in range(NB):
                i = i2 * NB + b
                bn = (b + 1) % NB
                wait_gather(i, b)
                fire_idx(lax.rem(i + 2, nck), b)
                wait_scatter(lax.rem(i + nck - 1, nck), bn)
                wait_idx(lax.rem(i + 1, nck), bn)
                fire_gather(lax.rem(i + 1, nck), bn)

                def ebody(e, ecarry):
                    wb = wrepv[b][e >> 3, pl.ds((e & 7) * 16, 16)]
                    for j in range(F // 16):
                        rows[b][e, pl.ds(j * 16, 16)] = rows[b][e, pl.ds(j * 16, 16)] * wb
                    return ecarry

                lax.fori_loop(0, C, ebody, 0)
                fire_scatter(i, b)
            return carry

        lax.fori_loop(0, jnp.where(fast, NCKF // NB, NCKS // NB), outer, 0)
        wait_scatter(nck - 1, 1)
        wait_gather(0, 0)
        wait_idx(1, 1)
        plsc.subcore_barrier()
        pltpu.sync_copy(acc.at[pl.ds(s * RPT, RPT)], out.at[c, pl.ds(s * RPT, RPT)])

    return pl.kernel(
        body,
        out_type=jax.ShapeDtypeStruct((NC, NPAD, F), jnp.float32),
        mesh=_MESH,
        scratch_types=[
            pltpu.VMEM((NCKMAX, C), jnp.int32),
        ] + [pltpu.VMEM((C,), jnp.int32)] * NB
          + [pltpu.VMEM((16, 128), jnp.float32)] * NB
          + [pltpu.VMEM((C, F), jnp.float32)] * NB
          + [pltpu.VMEM_SHARED((NPAD, F), jnp.float32)]
          + [pltpu.SemaphoreType.DMA] * (4 * NB),
        compiler_params=_SC_PARAMS,
    )


def _make_edge1_kernel():
    def body(src2, dst2, g, zf, out, srcv, dstv, *scr):
        rows = scr[0:NBUF]
        acc = scr[NBUF]
        semg = scr[NBUF + 1:NBUF + 1 + NBUF]
        sems = scr[NBUF + 1 + NBUF:NBUF + 1 + 2 * NBUF]
        c = lax.axis_index("c")
        s = lax.axis_index("s")
        wid = c * NS + s
        pltpu.sync_copy(src2.at[pl.ds(wid * NCHUNK, NCHUNK)], srcv)
        pltpu.sync_copy(dst2.at[pl.ds(wid * NCHUNK, NCHUNK)], dstv)
        pltpu.sync_copy(zf, acc.at[pl.ds(s * RPT, RPT)])

        pltpu.async_copy(g.at[srcv.at[0]], rows[0], semg[0])
        pltpu.async_copy(g.at[srcv.at[1]], rows[1], semg[1])
        plsc.subcore_barrier()

        def outer(i2, carry):
            for b in range(NBUF):
                i = i2 * NBUF + b
                bn = (b + 2) % NBUF

                @pl.when(i + 2 < NCHUNK)
                def _():
                    @pl.when(i >= 2)
                    def _():
                        pltpu.make_async_copy(rows[bn], acc.at[dstv.at[i - 2]], sems[bn]).wait()
                    pltpu.async_copy(g.at[srcv.at[i + 2]], rows[bn], semg[bn])

                pltpu.make_async_copy(g.at[srcv.at[i]], rows[b], semg[b]).wait()
                pltpu.async_copy(rows[b], acc.at[dstv.at[i]], sems[b], add=True)
            return carry

        lax.fori_loop(0, NCHUNK // NBUF, outer, 0)
        for j in range(NCHUNK - NBUF, NCHUNK):
            b = j % NBUF
            pltpu.make_async_copy(rows[b], acc.at[dstv.at[j]], sems[b]).wait()
        plsc.subcore_barrier()
        pltpu.sync_copy(acc.at[pl.ds(s * RPT, RPT)], out.at[pl.ds(c * NPAD + s * RPT, RPT)])

    return pl.kernel(
        body,
        out_type=jax.ShapeDtypeStruct((NC * NPAD,), jnp.float32),
        mesh=_MESH,
        scratch_types=[
            pltpu.VMEM((NCHUNK, C), jnp.int32),
            pltpu.VMEM((NCHUNK, C), jnp.int32),
        ] + [pltpu.VMEM((C,), jnp.float32)] * NBUF
          + [pltpu.VMEM_SHARED((NPAD,), jnp.float32)]
          + [pltpu.SemaphoreType.DMA] * (2 * NBUF),
        compiler_params=_SC_PARAMS,
    )


_deg_kernel = _make_deg_kernel()
_edge128 = _make_edge128_kernel()
_edge1 = _make_edge1_kernel()


def _tc_prep(xp, obsp, Wx, We, be, W0p, dw0, dw1, dc0, dc1):
    def body(x_ref, o_ref, wx_ref, we_ref, be_ref, w0_ref,
             dw0_ref, dw1_ref, dc0_ref, dc1_ref,
             u0_ref, g0_ref, dw_ref, d1_ref):
        h0 = (jnp.dot(x_ref[...], wx_ref[...], preferred_element_type=jnp.float32)
              + jnp.dot(o_ref[...], we_ref[...], preferred_element_type=jnp.float32)
              + be_ref[...])
        degw = dw0_ref[...] + dw1_ref[...] + 1.0
        deg1 = dc0_ref[...] + dc1_ref[...] + 1.0
        dw = lax.rsqrt(degw)
        d1 = lax.rsqrt(deg1)
        u0 = jnp.dot(h0, w0_ref[...], preferred_element_type=jnp.float32)
        u0_ref[...] = u0
        g0_ref[...] = u0 * dw
        dw_ref[...] = dw
        d1_ref[...] = d1

    return pl.pallas_call(
        body,
        out_shape=[
            jax.ShapeDtypeStruct((NPAD, HID), jnp.float32),
            jax.ShapeDtypeStruct((NPAD, HID), jnp.float32),
            jax.ShapeDtypeStruct((NPAD, 1), jnp.float32),
            jax.ShapeDtypeStruct((NPAD, 1), jnp.float32),
        ],
    )(xp, obsp, Wx, We, be, W0p, dw0, dw1, dc0, dc1)


def _tc_mid():
    def body(acc_ref, u_ref, dv_ref, b_ref, a_ref, w_ref, un_ref, gn_ref):
        acc = acc_ref[...]
        dv = dv_ref[...]
        t = dv * (acc[0] + acc[1]) + dv * dv * u_ref[...] + b_ref[...]
        a = a_ref[0, 0]
        hn = jnp.where(t >= 0, t, a * t)
        un = jnp.dot(hn, w_ref[...], preferred_element_type=jnp.float32)
        un_ref[...] = un
        gn_ref[...] = un * dv

    return pl.pallas_call(
        body,
        grid=(NBLK,),
        in_specs=[
            pl.BlockSpec((2, BR, HID), lambda i: (0, i, 0)),
            pl.BlockSpec((BR, HID), lambda i: (i, 0)),
            pl.BlockSpec((BR, 1), lambda i: (i, 0)),
            pl.BlockSpec((1, HID), lambda i: (0, 0)),
            pl.BlockSpec((1, 1), lambda i: (0, 0)),
            pl.BlockSpec((HID, HID), lambda i: (0, 0)),
        ],
        out_specs=[
            pl.BlockSpec((BR, HID), lambda i: (i, 0)),
            pl.BlockSpec((BR, HID), lambda i: (i, 0)),
        ],
        out_shape=[
            jax.ShapeDtypeStruct((NPAD, HID), jnp.float32),
            jax.ShapeDtypeStruct((NPAD, HID), jnp.float32),
        ],
    )


_tc_mid_call = _tc_mid()


def _tc_l3(acc, u3, dinvw, b3, a, W4, dinv1):
    def body(acc_ref, u_ref, dv_ref, b_ref, a_ref, w4_ref, d1_ref, u4_ref, g4_ref):
        accv = acc_ref[...]
        dv = dv_ref[...]
        t = dv * (accv[0] + accv[1]) + dv * dv * u_ref[...] + b_ref[...]
        av = a_ref[0, 0]
        h4 = jnp.where(t >= 0, t, av * t)
        u4 = jnp.dot(h4, w4_ref[...], preferred_element_type=jnp.float32)
        u4_ref[...] = u4
        g4_ref[...] = u4 * d1_ref[...]

    return pl.pallas_call(
        body,
        grid=(NBLK,),
        in_specs=[
            pl.BlockSpec((2, BR, HID), lambda i: (0, i, 0)),
            pl.BlockSpec((BR, HID), lambda i: (i, 0)),
            pl.BlockSpec((BR, 1), lambda i: (i, 0)),
            pl.BlockSpec((1, HID), lambda i: (0, 0)),
            pl.BlockSpec((1, 1), lambda i: (0, 0)),
            pl.BlockSpec((HID, 1), lambda i: (0, 0)),
            pl.BlockSpec((BR, 1), lambda i: (i, 0)),
        ],
        out_specs=[
            pl.BlockSpec((BR, 1), lambda i: (i, 0)),
            pl.BlockSpec((BR, 1), lambda i: (i, 0)),
        ],
        out_shape=[
            jax.ShapeDtypeStruct((NPAD, 1), jnp.float32),
            jax.ShapeDtypeStruct((NPAD, 1), jnp.float32),
        ],
    )(acc, u3, dinvw, b3, a, W4, dinv1)


def _tc_final(acc4, u4, dinv1, b4, ids):
    def body(acc_ref, u_ref, d1_ref, b_ref, ids_ref, out_ref, sums_ref):
        i = pl.program_id(0)

        @pl.when(i == 0)
        def _():
            sums_ref[...] = jnp.zeros((2, G), jnp.float32)

        accv = acc_ref[...]
        d1 = d1_ref[...]
        z = d1 * (accv[0] + accv[1]) + d1 * d1 * u_ref[...] + b_ref[0, 0]
        iota = lax.broadcasted_iota(jnp.int32, (1, G), 1)
        m = (ids_ref[...] == iota).astype(jnp.float32)
        sums_ref[0:1, :] += jnp.sum(m * z, axis=0, keepdims=True)
        sums_ref[1:2, :] += jnp.sum(m, axis=0, keepdims=True)

        @pl.when(i == NBLK - 1)
        def _():
            out_ref[...] = sums_ref[0:1, :] / jnp.maximum(sums_ref[1:2, :], 1.0)

    return pl.pallas_call(
        body,
        grid=(NBLK,),
        in_specs=[
            pl.BlockSpec((2, BR, 1), lambda i: (0, i, 0)),
            pl.BlockSpec((BR, 1), lambda i: (i, 0)),
            pl.BlockSpec((BR, 1), lambda i: (i, 0)),
            pl.BlockSpec((1, 1), lambda i: (0, 0)),
            pl.BlockSpec((BR, 1), lambda i: (i, 0)),
        ],
        out_specs=pl.BlockSpec((1, G), lambda i: (0, 0)),
        out_shape=jax.ShapeDtypeStruct((1, G), jnp.float32),
        scratch_shapes=[pltpu.VMEM((2, G), jnp.float32)],
    )(acc4, u4, dinv1, b4, ids)


def kernel(x, observation, edge_index, edge_weight, batch_ids,
           W_emb, b_emb, W0, b0, W1, b1, W2, b2, W3, b3, W4, b4, prelu_a):
    f32 = jnp.float32
    src = edge_index[0]
    dst = edge_index[1]
    npad_e = EPAD - E

    srcp = jnp.concatenate([src, jnp.full((npad_e,), N, jnp.int32)]).reshape(EPAD // C, C)
    dstp = jnp.concatenate([dst, jnp.full((npad_e,), N, jnp.int32)]).reshape(EPAD // C, C)
    wp = jnp.concatenate([edge_weight, jnp.zeros((npad_e,), f32)])
    w2 = wp.reshape(EPAD // C, C)
    wrep = jnp.broadcast_to(wp[:, None], (EPAD, 16)).reshape(EPAD * 16 // C, C)

    xp = jnp.pad(x, ((0, NPAD - N), (0, 0)))
    obsp = jnp.pad(observation, ((0, NPAD - N), (0, 0)))
    ids = jnp.pad(batch_ids, (0, NPAD - N), constant_values=G).reshape(NPAD, 1)

    Wx = jnp.eye(4, 16, dtype=f32)
    We = jnp.zeros((6, 16), f32).at[:, 4:10].set(W_emb)
    be = jnp.zeros((1, 16), f32).at[0, 4:10].set(b_emb)
    W0p = jnp.zeros((16, HID), f32).at[0:10, :].set(W0)

    zf1 = jnp.zeros((RPT,), f32)

    a = prelu_a.reshape(1, 1)

    degf = _deg_kernel(dstp, w2, zf1)
    dw0 = degf[0 * NPAD:1 * NPAD].reshape(NPAD, 1)
    dc0 = degf[1 * NPAD:2 * NPAD].reshape(NPAD, 1)
    dw1 = degf[2 * NPAD:3 * NPAD].reshape(NPAD, 1)
    dc1 = degf[3 * NPAD:4 * NPAD].reshape(NPAD, 1)
    u0, g0, dinvw, dinv1 = _tc_prep(xp, obsp, Wx, We, be, W0p, dw0, dw1, dc0, dc1)

    acc0 = _edge128(srcp, dstp, wrep, g0)
    u1, g1 = _tc_mid_call(acc0, u0, dinvw, b0.reshape(1, HID), a, W1)

    acc1 = _edge128(srcp, dstp, wrep, g1)
    u2, g2 = _tc_mid_call(acc1, u1, dinvw, b1.reshape(1, HID), a, W2)

    acc2 = _edge128(srcp, dstp, wrep, g2)
    u3, g3 = _tc_mid_call(acc2, u2, dinvw, b2.reshape(1, HID), a, W3)

    acc3 = _edge128(srcp, dstp, wrep, g3)
    u4, g4 = _tc_l3(acc3, u3, dinvw, b3.reshape(1, HID), a, W4, dinv1)

    acc4 = _edge1(srcp, dstp, g4.reshape(NPAD), zf1).reshape(NC, NPAD, 1)
    pooled = _tc_final(acc4, u4, dinv1, b4.reshape(1, 1), ids)
    return pooled.reshape(-1)
